# baseline (speedup 1.0000x reference)
import functools

import jax
import jax.numpy as jnp
from jax.experimental import pallas as pl
from jax.experimental.pallas import tpu as pltpu

DIM = 1024
HIDDEN = 4096
NUM_EXPERTS = 8
TOP_K = 2
HT = 2048
N_HT = HIDDEN // HT
N_STEPS = NUM_EXPERTS * N_HT
N_TOK = 64
NBUF = 3


def _moe_body(x_ref, wr_ref, w1_hbm, w2_hbm, out_ref,
              xt_s, coeff_s, acc_s, w1_buf, w2_buf, sem1, sem2):

    g = pl.program_id(0)

    def w1_copy(step, buf):
        e, h = step // N_HT, step % N_HT
        return pltpu.make_async_copy(
            w1_hbm.at[e, pl.ds(h * HT, HT), :], w1_buf.at[buf], sem1.at[buf])

    def w2_copy(step, buf):
        e, h = step // N_HT, step % N_HT
        return pltpu.make_async_copy(
            w2_hbm.at[e, :, pl.ds(h * HT, HT)], w2_buf.at[buf], sem2.at[buf])

    @pl.when(g == 0)
    def _prime():
        for s in range(NBUF):
            w1_copy(s, s).start()
            w2_copy(s, s).start()

    @pl.when(g == 0)
    def _prologue():
        xt = x_ref[...].T
        xt_s[...] = xt
        logits = jnp.dot(wr_ref[...], xt, preferred_element_type=jnp.float32)
        m = jnp.max(logits, axis=0, keepdims=True)
        p = jnp.exp(logits - m)
        p = p / jnp.sum(p, axis=0, keepdims=True)

        iota = jax.lax.broadcasted_iota(jnp.int32, p.shape, 0)
        big = jnp.int32(NUM_EXPERTS)
        m1 = jnp.max(p, axis=0, keepdims=True)
        idx1 = jnp.min(jnp.where(p == m1, iota, big), axis=0, keepdims=True)
        mask1 = iota == idx1
        p2 = jnp.where(mask1, -1.0, p)
        m2 = jnp.max(p2, axis=0, keepdims=True)
        idx2 = jnp.min(jnp.where(p2 == m2, iota, big), axis=0, keepdims=True)
        mask2 = iota == idx2
        coeff_s[...] = jnp.where(mask1 | mask2, p, 0.0) / (m1 + m2)

    buf = jax.lax.rem(g, NBUF)

    e = g // N_HT
    xt = xt_s[...]
    coeff = coeff_s[...]
    iota_e = jax.lax.broadcasted_iota(jnp.int32, coeff.shape, 0)
    coeff_e = jnp.sum(jnp.where(iota_e == e, coeff, 0.0), axis=0, keepdims=True)

    w1_copy(g, buf).wait()
    h1 = jnp.dot(w1_buf[buf], xt, preferred_element_type=jnp.float32)
    h1 = h1 * jax.nn.sigmoid(h1)
    w2_copy(g, buf).wait()
    part = jnp.dot(w2_buf[buf], h1, preferred_element_type=jnp.float32) * coeff_e

    @pl.when(g == 0)
    def _init():
        acc_s[...] = part

    @pl.when(g > 0)
    def _acc():
        acc_s[...] += part

    @pl.when(g < N_STEPS - NBUF)
    def _refill():
        w1_copy(g + NBUF, buf).start()
        w2_copy(g + NBUF, buf).start()

    @pl.when(g == N_STEPS - 1)
    def _epilogue():
        out_ref[...] = acc_s[...].T


@functools.partial(jax.jit, static_argnames=())
def kernel(x, Wr, W1, W2):
    b, s, d = x.shape
    n_tok = b * s
    x_flat = x.reshape(n_tok, d)

    out = pl.pallas_call(
        _moe_body,
        grid=(N_STEPS,),
        in_specs=[
            pl.BlockSpec((n_tok, d), lambda g: (0, 0)),
            pl.BlockSpec((NUM_EXPERTS, d), lambda g: (0, 0)),
            pl.BlockSpec(memory_space=pltpu.MemorySpace.HBM),
            pl.BlockSpec(memory_space=pltpu.MemorySpace.HBM),
        ],
        out_specs=pl.BlockSpec((n_tok, d), lambda g: (0, 0)),
        out_shape=jax.ShapeDtypeStruct((n_tok, d), jnp.float32),
        scratch_shapes=[
            pltpu.VMEM((d, n_tok), jnp.float32),
            pltpu.VMEM((NUM_EXPERTS, n_tok), jnp.float32),
            pltpu.VMEM((d, n_tok), jnp.float32),
            pltpu.VMEM((NBUF, HT, DIM), jnp.float32),
            pltpu.VMEM((NBUF, DIM, HT), jnp.float32),
            pltpu.SemaphoreType.DMA((NBUF,)),
            pltpu.SemaphoreType.DMA((NBUF,)),
        ],
        compiler_params=pltpu.CompilerParams(
            dimension_semantics=("arbitrary",),
        ),
    )(x_flat, Wr, W1, W2)

    return out.reshape(b, s, d)


# speedup vs baseline: 1.0090x; 1.0090x over previous
import functools

import jax
import jax.numpy as jnp
from jax.experimental import pallas as pl
from jax.experimental.pallas import tpu as pltpu

DIM = 1024
HIDDEN = 4096
NUM_EXPERTS = 8
TOP_K = 2
HT = 2048
N_HT = HIDDEN // HT
N_STEPS = NUM_EXPERTS * N_HT
N_TOK = 64
NBUF = 3


def _moe_body(x_ref, wr_ref, w1_hbm, w2_hbm, out_ref,
              xt_s, coeff_s, acc_s, w1_buf, w2_buf, sem1, sem2):

    g = pl.program_id(0)

    def w1_copy(step, buf):
        e, h = step // N_HT, step % N_HT
        return pltpu.make_async_copy(
            w1_hbm.at[e, pl.ds(h * HT, HT), :], w1_buf.at[buf], sem1.at[buf])

    def w2_copy(step, buf):
        e, h = step // N_HT, step % N_HT
        return pltpu.make_async_copy(
            w2_hbm.at[e, :, pl.ds(h * HT, HT)], w2_buf.at[buf], sem2.at[buf])

    @pl.when(g == 0)
    def _prime():
        for s in range(NBUF - 1):
            w1_copy(s, s).start()
            w2_copy(s, s).start()

    nxt = g + NBUF - 1

    @pl.when(nxt < N_STEPS)
    def _refill():
        nbuf = jax.lax.rem(nxt, NBUF)
        w1_copy(nxt, nbuf).start()
        w2_copy(nxt, nbuf).start()

    @pl.when(g == 0)
    def _prologue():
        xt = x_ref[...].T
        xt_s[...] = xt
        logits = jnp.dot(wr_ref[...], xt, preferred_element_type=jnp.float32)
        m = jnp.max(logits, axis=0, keepdims=True)
        p = jnp.exp(logits - m)
        p = p / jnp.sum(p, axis=0, keepdims=True)

        iota = jax.lax.broadcasted_iota(jnp.int32, p.shape, 0)
        big = jnp.int32(NUM_EXPERTS)
        m1 = jnp.max(p, axis=0, keepdims=True)
        idx1 = jnp.min(jnp.where(p == m1, iota, big), axis=0, keepdims=True)
        mask1 = iota == idx1
        p2 = jnp.where(mask1, -1.0, p)
        m2 = jnp.max(p2, axis=0, keepdims=True)
        idx2 = jnp.min(jnp.where(p2 == m2, iota, big), axis=0, keepdims=True)
        mask2 = iota == idx2
        coeff_s[...] = jnp.where(mask1 | mask2, p, 0.0) / (m1 + m2)

    buf = jax.lax.rem(g, NBUF)

    e = g // N_HT
    xt = xt_s[...]
    coeff = coeff_s[...]
    iota_e = jax.lax.broadcasted_iota(jnp.int32, coeff.shape, 0)
    coeff_e = jnp.sum(jnp.where(iota_e == e, coeff, 0.0), axis=0, keepdims=True)

    w1_copy(g, buf).wait()
    w2_copy(g, buf).wait()
    h1 = jnp.dot(w1_buf[buf], xt, preferred_element_type=jnp.float32)
    h1 = h1 * jax.nn.sigmoid(h1)
    part = jnp.dot(w2_buf[buf], h1, preferred_element_type=jnp.float32) * coeff_e

    @pl.when(g == 0)
    def _init():
        acc_s[...] = part

    @pl.when(g > 0)
    def _acc():
        acc_s[...] += part

    @pl.when(g == N_STEPS - 1)
    def _epilogue():
        out_ref[...] = acc_s[...].T


@functools.partial(jax.jit, static_argnames=())
def kernel(x, Wr, W1, W2):
    b, s, d = x.shape
    n_tok = b * s
    x_flat = x.reshape(n_tok, d)

    out = pl.pallas_call(
        _moe_body,
        grid=(N_STEPS,),
        in_specs=[
            pl.BlockSpec((n_tok, d), lambda g: (0, 0)),
            pl.BlockSpec((NUM_EXPERTS, d), lambda g: (0, 0)),
            pl.BlockSpec(memory_space=pltpu.MemorySpace.HBM),
            pl.BlockSpec(memory_space=pltpu.MemorySpace.HBM),
        ],
        out_specs=pl.BlockSpec((n_tok, d), lambda g: (0, 0)),
        out_shape=jax.ShapeDtypeStruct((n_tok, d), jnp.float32),
        scratch_shapes=[
            pltpu.VMEM((d, n_tok), jnp.float32),
            pltpu.VMEM((NUM_EXPERTS, n_tok), jnp.float32),
            pltpu.VMEM((d, n_tok), jnp.float32),
            pltpu.VMEM((NBUF, HT, DIM), jnp.float32),
            pltpu.VMEM((NBUF, DIM, HT), jnp.float32),
            pltpu.SemaphoreType.DMA((NBUF,)),
            pltpu.SemaphoreType.DMA((NBUF,)),
        ],
        compiler_params=pltpu.CompilerParams(
            dimension_semantics=("arbitrary",),
        ),
    )(x_flat, Wr, W1, W2)

    return out.reshape(b, s, d)
